# speedup vs baseline: 1.2860x; 1.2860x over previous
import functools

import jax
import jax.numpy as jnp
from jax import lax
from jax.experimental import pallas as pl
from jax.experimental.pallas import tpu as pltpu
from jax.experimental.pallas import tpu_sc as plsc

B = 16384
D = 128
NUM_CORES = 2
NUM_SUBCORES = 16
NW = NUM_CORES * NUM_SUBCORES
B_PER_W = B // NW
CHUNK = 128
NCHUNK = B_PER_W // CHUNK
LANES = 16
VPR = D // LANES

_mesh = plsc.VectorSubcoreMesh(core_axis_name="c", subcore_axis_name="s")


@functools.partial(
    pl.kernel,
    mesh=_mesh,
    out_type=jax.ShapeDtypeStruct((B, D), jnp.float32),
    scratch_types=[
        pltpu.VMEM((B_PER_W,), jnp.int32),
        pltpu.VMEM((B_PER_W,), jnp.int32),
        pltpu.VMEM((CHUNK, D), jnp.float32),
        pltpu.VMEM((CHUNK, D), jnp.float32),
        pltpu.SemaphoreType.DMA,
        pltpu.SemaphoreType.DMA,
    ],
)
def _gmf_sc(uidx_hbm, iidx_hbm, utab_hbm, itab_hbm, out_hbm,
            uidx_v, iidx_v, urows_v, irows_v, usem, isem):
    wid = lax.axis_index("s") * NUM_CORES + lax.axis_index("c")
    base = wid * B_PER_W
    pltpu.sync_copy(uidx_hbm.at[pl.ds(base, B_PER_W)], uidx_v)
    pltpu.sync_copy(iidx_hbm.at[pl.ds(base, B_PER_W)], iidx_v)

    for ci in range(NCHUNK):
        cbase = ci * CHUNK
        cu = pltpu.async_copy(
            utab_hbm.at[uidx_v.at[pl.ds(cbase, CHUNK)]], urows_v, usem)
        cit = pltpu.async_copy(
            itab_hbm.at[iidx_v.at[pl.ds(cbase, CHUNK)]], irows_v, isem)
        cu.wait()
        cit.wait()

        def row_body(r, _):
            for v in range(VPR):
                sl = pl.ds(v * LANES, LANES)
                urows_v[r, sl] = urows_v[r, sl] * irows_v[r, sl]
            return 0

        lax.fori_loop(0, CHUNK, row_body, 0)
        pltpu.sync_copy(urows_v, out_hbm.at[pl.ds(base + cbase, CHUNK)])


def kernel(user_indices, item_indices, user_table, item_table):
    return _gmf_sc(
        user_indices.astype(jnp.int32),
        item_indices.astype(jnp.int32),
        user_table,
        item_table,
    )


# speedup vs baseline: 1.4493x; 1.1269x over previous
import functools

import jax
import jax.numpy as jnp
from jax import lax
from jax.experimental import pallas as pl
from jax.experimental.pallas import tpu as pltpu
from jax.experimental.pallas import tpu_sc as plsc

B = 16384
D = 128
NUM_CORES = 2
NUM_SUBCORES = 16
NW = NUM_CORES * NUM_SUBCORES
B_PER_W = B // NW
CHUNK = 128
NCHUNK = B_PER_W // CHUNK
LANES = 16
VPR = D // LANES
NBUF = 2

_mesh = plsc.VectorSubcoreMesh(core_axis_name="c", subcore_axis_name="s")


@functools.partial(
    pl.kernel,
    mesh=_mesh,
    out_type=jax.ShapeDtypeStruct((B, D), jnp.float32),
    scratch_types=[
        pltpu.VMEM((B_PER_W,), jnp.int32),
        pltpu.VMEM((B_PER_W,), jnp.int32),
        pltpu.VMEM((NBUF, CHUNK, D), jnp.float32),
        pltpu.VMEM((NBUF, CHUNK, D), jnp.float32),
        pltpu.VMEM((NBUF, CHUNK, D), jnp.float32),
        pltpu.SemaphoreType.DMA((NBUF,)),
        pltpu.SemaphoreType.DMA((NBUF,)),
        pltpu.SemaphoreType.DMA((NBUF,)),
    ],
)
def _gmf_sc(uidx_hbm, iidx_hbm, utab_hbm, itab_hbm, out_hbm,
            uidx_v, iidx_v, ubuf, ibuf, obuf, usem, isem, osem):
    wid = lax.axis_index("s") * NUM_CORES + lax.axis_index("c")
    base = wid * B_PER_W
    pltpu.sync_copy(uidx_hbm.at[pl.ds(base, B_PER_W)], uidx_v)
    pltpu.sync_copy(iidx_hbm.at[pl.ds(base, B_PER_W)], iidx_v)

    def start_gathers(ci):
        b = ci % NBUF
        cbase = ci * CHUNK
        gu = pltpu.async_copy(
            utab_hbm.at[uidx_v.at[pl.ds(cbase, CHUNK)]], ubuf.at[b], usem.at[b])
        gi = pltpu.async_copy(
            itab_hbm.at[iidx_v.at[pl.ds(cbase, CHUNK)]], ibuf.at[b], isem.at[b])
        return gu, gi

    gathers = [start_gathers(0), start_gathers(1)]
    stores = [None] * NCHUNK

    for ci in range(NCHUNK):
        b = ci % NBUF
        gu, gi = gathers[ci]
        gu.wait()
        gi.wait()
        if ci - NBUF >= 0:
            stores[ci - NBUF].wait()

        def row_body(r, _):
            for v in range(VPR):
                sl = pl.ds(v * LANES, LANES)
                obuf[b, r, sl] = ubuf[b, r, sl] * ibuf[b, r, sl]
            return 0

        lax.fori_loop(0, CHUNK, row_body, 0)

        if ci + NBUF < NCHUNK:
            gathers.append(start_gathers(ci + NBUF))
        stores[ci] = pltpu.async_copy(
            obuf.at[b], out_hbm.at[pl.ds(base + ci * CHUNK, CHUNK)], osem.at[b])

    for ci in range(NCHUNK - NBUF, NCHUNK):
        stores[ci].wait()


def kernel(user_indices, item_indices, user_table, item_table):
    return _gmf_sc(
        user_indices.astype(jnp.int32),
        item_indices.astype(jnp.int32),
        user_table,
        item_table,
    )


# speedup vs baseline: 1.5038x; 1.0376x over previous
import functools

import jax
import jax.numpy as jnp
from jax import lax
from jax.experimental import pallas as pl
from jax.experimental.pallas import tpu as pltpu
from jax.experimental.pallas import tpu_sc as plsc

B = 16384
D = 128
NUM_CORES = 2
NUM_SUBCORES = 16
NW = NUM_CORES * NUM_SUBCORES
B_PER_W = B // NW
CHUNK = 64
NCHUNK = B_PER_W // CHUNK
LANES = 16
VPR = D // LANES
NBUF = 4

_mesh = plsc.VectorSubcoreMesh(core_axis_name="c", subcore_axis_name="s")


@functools.partial(
    pl.kernel,
    mesh=_mesh,
    out_type=jax.ShapeDtypeStruct((B, D), jnp.float32),
    scratch_types=[
        pltpu.VMEM((B_PER_W,), jnp.int32),
        pltpu.VMEM((B_PER_W,), jnp.int32),
        pltpu.VMEM((NBUF, CHUNK, D), jnp.float32),
        pltpu.VMEM((NBUF, CHUNK, D), jnp.float32),
        pltpu.VMEM((NBUF, CHUNK, D), jnp.float32),
        pltpu.SemaphoreType.DMA((NBUF,)),
        pltpu.SemaphoreType.DMA((NBUF,)),
        pltpu.SemaphoreType.DMA((NBUF,)),
    ],
)
def _gmf_sc(uidx_hbm, iidx_hbm, utab_hbm, itab_hbm, out_hbm,
            uidx_v, iidx_v, ubuf, ibuf, obuf, usem, isem, osem):
    wid = lax.axis_index("s") * NUM_CORES + lax.axis_index("c")
    base = wid * B_PER_W
    pltpu.sync_copy(uidx_hbm.at[pl.ds(base, B_PER_W)], uidx_v)
    pltpu.sync_copy(iidx_hbm.at[pl.ds(base, B_PER_W)], iidx_v)

    def start_gathers(ci):
        b = ci % NBUF
        cbase = ci * CHUNK
        gu = pltpu.async_copy(
            utab_hbm.at[uidx_v.at[pl.ds(cbase, CHUNK)]], ubuf.at[b], usem.at[b])
        gi = pltpu.async_copy(
            itab_hbm.at[iidx_v.at[pl.ds(cbase, CHUNK)]], ibuf.at[b], isem.at[b])
        return gu, gi

    gathers = [start_gathers(ci) for ci in range(NBUF)]
    stores = [None] * NCHUNK

    for ci in range(NCHUNK):
        b = ci % NBUF
        gu, gi = gathers[ci]
        gu.wait()
        gi.wait()
        if ci - NBUF >= 0:
            stores[ci - NBUF].wait()

        def row_body(r, _):
            for v in range(VPR):
                sl = pl.ds(v * LANES, LANES)
                obuf[b, r, sl] = ubuf[b, r, sl] * ibuf[b, r, sl]
            return 0

        lax.fori_loop(0, CHUNK, row_body, 0)

        if ci + NBUF < NCHUNK:
            gathers.append(start_gathers(ci + NBUF))
        stores[ci] = pltpu.async_copy(
            obuf.at[b], out_hbm.at[pl.ds(base + ci * CHUNK, CHUNK)], osem.at[b])

    for ci in range(NCHUNK - NBUF, NCHUNK):
        stores[ci].wait()


def kernel(user_indices, item_indices, user_table, item_table):
    return _gmf_sc(
        user_indices.astype(jnp.int32),
        item_indices.astype(jnp.int32),
        user_table,
        item_table,
    )


# speedup vs baseline: 1.5348x; 1.0206x over previous
import functools

import jax
import jax.numpy as jnp
from jax import lax
from jax.experimental import pallas as pl
from jax.experimental.pallas import tpu as pltpu
from jax.experimental.pallas import tpu_sc as plsc

B = 16384
D = 128
NUM_CORES = 2
NUM_SUBCORES = 16
NW = NUM_CORES * NUM_SUBCORES
B_PER_W = B // NW
CHUNK = 64
NCHUNK = B_PER_W // CHUNK
LANES = 16
VPR = D // LANES
NBUF = 4

_mesh = plsc.VectorSubcoreMesh(core_axis_name="c", subcore_axis_name="s")


@functools.partial(
    pl.kernel,
    mesh=_mesh,
    out_type=jax.ShapeDtypeStruct((B, D), jnp.float32),
    scratch_types=[
        pltpu.VMEM((B_PER_W,), jnp.int32),
        pltpu.VMEM((B_PER_W,), jnp.int32),
        pltpu.VMEM((NBUF, CHUNK, D), jnp.float32),
        pltpu.VMEM((NBUF, CHUNK, D), jnp.float32),
        pltpu.VMEM((NBUF, CHUNK, D), jnp.float32),
        pltpu.SemaphoreType.DMA((NBUF,)),
        pltpu.SemaphoreType.DMA((NBUF,)),
        pltpu.SemaphoreType.DMA((NBUF,)),
        pltpu.SemaphoreType.DMA,
    ],
)
def _gmf_sc(uidx_hbm, iidx_hbm, utab_hbm, itab_hbm, out_hbm,
            uidx_v, iidx_v, ubuf, ibuf, obuf, usem, isem, osem, xsem):
    wid = lax.axis_index("s") * NUM_CORES + lax.axis_index("c")
    base = wid * B_PER_W
    lu = pltpu.async_copy(uidx_hbm.at[pl.ds(base, B_PER_W)], uidx_v, xsem)
    li = pltpu.async_copy(iidx_hbm.at[pl.ds(base, B_PER_W)], iidx_v, xsem)
    lu.wait()
    li.wait()

    def start_gathers(ci):
        b = ci % NBUF
        cbase = ci * CHUNK
        gu = pltpu.async_copy(
            utab_hbm.at[uidx_v.at[pl.ds(cbase, CHUNK)]], ubuf.at[b], usem.at[b])
        gi = pltpu.async_copy(
            itab_hbm.at[iidx_v.at[pl.ds(cbase, CHUNK)]], ibuf.at[b], isem.at[b])
        return gu, gi

    gathers = [start_gathers(ci) for ci in range(NBUF)]
    stores = [None] * NCHUNK

    for ci in range(NCHUNK):
        b = ci % NBUF
        gu, gi = gathers[ci]
        gu.wait()
        gi.wait()
        if ci - NBUF >= 0:
            stores[ci - NBUF].wait()

        def row_body(r, _):
            for v in range(VPR):
                sl = pl.ds(v * LANES, LANES)
                obuf[b, r, sl] = ubuf[b, r, sl] * ibuf[b, r, sl]
            return 0

        lax.fori_loop(0, CHUNK, row_body, 0)

        if ci + NBUF < NCHUNK:
            gathers.append(start_gathers(ci + NBUF))
        stores[ci] = pltpu.async_copy(
            obuf.at[b], out_hbm.at[pl.ds(base + ci * CHUNK, CHUNK)], osem.at[b])

    for ci in range(NCHUNK - NBUF, NCHUNK):
        stores[ci].wait()


def kernel(user_indices, item_indices, user_table, item_table):
    return _gmf_sc(
        user_indices.astype(jnp.int32),
        item_indices.astype(jnp.int32),
        user_table,
        item_table,
    )
